# baseline (speedup 1.0000x reference)
import functools

import jax
import jax.numpy as jnp
from jax import lax
from jax.experimental import pallas as pl
from jax.experimental.pallas import tpu as pltpu
import jax.experimental.pallas.tpu_sc as plsc

B, L, NF = 4096, 50, 4
VOCAB, DIM = 100000, 32
OUT_DIM = NF * DIM + 3
N = B * L
LANES = 128
NROWS = N // LANES
NW = 32
GROUP = 5
CHUNK_TOK = GROUP * LANES

SLICES = 2
N_S = N // SLICES
ROWS_S_PER_W = NROWS // SLICES // NW
TOK_S_PER_W = N_S // NW
NCHUNK = ROWS_S_PER_W // GROUP


def _sc_gather_body(t0, t1, t2, t3, idx, g_hbm, idx_v, buf_v, sem):
    tables = (t0, t1, t2, t3)
    wid = lax.axis_index("s") * 2 + lax.axis_index("c")
    base_tok = wid * TOK_S_PER_W

    for f in range(NF):
        pltpu.sync_copy(idx.at[f, wid], idx_v.at[f])

    def chunk(c, carry):
        copies = []
        for f in range(NF):
            for j in range(GROUP):
                r = c * GROUP + j
                copies.append(
                    pltpu.async_copy(
                        tables[f].at[idx_v.at[f, r]],
                        buf_v.at[f, pl.ds(j * LANES, LANES)],
                        sem,
                    )
                )
        for cp in copies:
            cp.wait()
        for f in range(NF):
            pltpu.sync_copy(
                buf_v.at[f],
                g_hbm.at[pl.ds(base_tok + c * CHUNK_TOK, CHUNK_TOK),
                         pl.ds(f * DIM, DIM)],
            )
        return carry

    lax.fori_loop(0, NCHUNK, chunk, 0)


_sc_gather = functools.partial(
    pl.kernel,
    mesh=plsc.VectorSubcoreMesh(core_axis_name="c", subcore_axis_name="s"),
    compiler_params=pltpu.CompilerParams(use_tc_tiling_on_sc=False),
    out_type=jax.ShapeDtypeStruct((N_S, NF * DIM), jnp.float32),
    scratch_types=[
        pltpu.VMEM((NF, ROWS_S_PER_W, LANES), jnp.int32),
        pltpu.VMEM((NF, CHUNK_TOK, DIM), jnp.float32),
        pltpu.SemaphoreType.DMA,
    ],
)(_sc_gather_body)


TILE = 1024
TILES_S = N_S // TILE


def _mlp_body(g_ref, x_ref, w1m_ref, w1e_ref, b1_ref, w2_ref, b2_ref,
              w3_ref, b3_ref, p_ref, prev_ref, o_ref):
    hp = jnp.float32
    g = g_ref[...]
    ex = x_ref[...]
    h = jnp.dot(g, w1m_ref[...], preferred_element_type=hp)
    h = h + jnp.dot(ex, w1e_ref[...], preferred_element_type=hp)
    h = jnp.maximum(h + b1_ref[...], 0.0)
    h = jnp.dot(h, w2_ref[...], preferred_element_type=hp) + b2_ref[...]
    h = jnp.maximum(h, 0.0)
    o = jnp.dot(h, w3_ref[...], preferred_element_type=hp) + b3_ref[...]
    o = o + jnp.dot(ex, p_ref[...], preferred_element_type=hp)
    o_ref[...] = o


def _mlp_slice(s, g, xr, w1m, w1e, b1, w2, b2, w3e, b3e, p, prev):
    off = s * TILES_S
    return pl.pallas_call(
        _mlp_body,
        grid=(TILES_S,),
        in_specs=[
            pl.BlockSpec((TILE, NF * DIM), lambda i: (i, 0)),
            pl.BlockSpec((TILE, NF + 3), lambda i: (i + off, 0)),
            pl.BlockSpec((NF * DIM, 128), lambda i: (0, 0)),
            pl.BlockSpec((NF + 3, 128), lambda i: (0, 0)),
            pl.BlockSpec((1, 128), lambda i: (0, 0)),
            pl.BlockSpec((128, 128), lambda i: (0, 0)),
            pl.BlockSpec((1, 128), lambda i: (0, 0)),
            pl.BlockSpec((128, OUT_DIM), lambda i: (0, 0)),
            pl.BlockSpec((1, OUT_DIM), lambda i: (0, 0)),
            pl.BlockSpec((NF + 3, OUT_DIM), lambda i: (0, 0)),
            pl.BlockSpec(memory_space=pl.ANY),
        ],
        out_specs=pl.BlockSpec((TILE, OUT_DIM), lambda i: (i + off, 0)),
        out_shape=jax.ShapeDtypeStruct((N, OUT_DIM), jnp.float32),
        input_output_aliases={10: 0},
    )(g, xr, w1m, w1e, b1, w2, b2, w3e, b3e, p, prev)


def kernel(x, table_0, table_1, table_2, table_3, W1, b1, W2, b2, W3, b3):
    xr = x.reshape(N, NF + 3)
    xi = xr[:, :NF].astype(jnp.int32)
    idx = xi.T.reshape(NF, SLICES, NW, ROWS_S_PER_W, LANES)

    gs = [
        _sc_gather(table_0, table_1, table_2, table_3, idx[:, s])
        for s in range(SLICES)
    ]

    w1m = W1[: NF * DIM]
    w1e = jnp.concatenate(
        [jnp.zeros((NF, 128), jnp.float32), W1[NF * DIM:]], axis=0)
    w3e = jnp.concatenate([W3, jnp.zeros((128, 1), jnp.float32)], axis=1)
    b3e = jnp.concatenate([b3, jnp.zeros((1,), jnp.float32)])[None, :]
    p = jnp.zeros((NF + 3, OUT_DIM), jnp.float32).at[NF + 2, OUT_DIM - 1].set(1.0)

    out = jnp.zeros((N, OUT_DIM), jnp.float32)
    for s in range(SLICES):
        out = _mlp_slice(s, gs[s], xr, w1m, w1e, b1[None, :], W2,
                         b2[None, :], w3e, b3e, p, out)
    return out.reshape(B, L, OUT_DIM)


# speedup vs baseline: 1.0735x; 1.0735x over previous
import functools

import jax
import jax.numpy as jnp
from jax import lax
from jax.experimental import pallas as pl
from jax.experimental.pallas import tpu as pltpu
import jax.experimental.pallas.tpu_sc as plsc

B, L, NF = 4096, 50, 4
VOCAB, DIM = 100000, 32
OUT_DIM = NF * DIM + 3
N = B * L
LANES = 128
NROWS = N // LANES
NW = 32
ROWS_PER_W = NROWS // NW
TOK_PER_W = N // NW
GROUP = 5
CHUNK_TOK = GROUP * LANES
NCHUNK = ROWS_PER_W // GROUP


def _sc_gather_body(t0, t1, t2, t3, idx, g_hbm, idx_v, buf_v, sem):
    tables = (t0, t1, t2, t3)
    wid = lax.axis_index("s") * 2 + lax.axis_index("c")
    base_tok = wid * TOK_PER_W

    for f in range(NF):
        pltpu.sync_copy(idx.at[f, pl.ds(wid * ROWS_PER_W, ROWS_PER_W)],
                        idx_v.at[f])

    def chunk(c, carry):
        copies = []
        for f in range(NF):
            for j in range(GROUP):
                r = c * GROUP + j
                copies.append(
                    pltpu.async_copy(
                        tables[f].at[idx_v.at[f, r]],
                        buf_v.at[f, pl.ds(j * LANES, LANES)],
                        sem,
                    )
                )
        for cp in copies:
            cp.wait()
        for f in range(NF):
            pltpu.sync_copy(
                buf_v.at[f],
                g_hbm.at[pl.ds(base_tok + c * CHUNK_TOK, CHUNK_TOK),
                         pl.ds(f * DIM, DIM)],
            )
        return carry

    lax.fori_loop(0, NCHUNK, chunk, 0)


_sc_gather = functools.partial(
    pl.kernel,
    mesh=plsc.VectorSubcoreMesh(core_axis_name="c", subcore_axis_name="s"),
    compiler_params=pltpu.CompilerParams(use_tc_tiling_on_sc=False),
    out_type=jax.ShapeDtypeStruct((N, NF * DIM), jnp.float32),
    scratch_types=[
        pltpu.VMEM((NF, ROWS_PER_W, LANES), jnp.int32),
        pltpu.VMEM((NF, CHUNK_TOK, DIM), jnp.float32),
        pltpu.SemaphoreType.DMA,
    ],
)(_sc_gather_body)


TILE = 1024
TILES = N // TILE


def _mlp_body(g_ref, x_ref, w1m_ref, w1e_ref, b1_ref, w2_ref, b2_ref,
              w3_ref, b3_ref, p_ref, o_ref):
    hp = jnp.float32
    g = g_ref[...]
    ex = x_ref[...]
    h = jnp.dot(g, w1m_ref[...], preferred_element_type=hp)
    h = h + jnp.dot(ex, w1e_ref[...], preferred_element_type=hp)
    h = jnp.maximum(h + b1_ref[...], 0.0)
    h = jnp.dot(h, w2_ref[...], preferred_element_type=hp) + b2_ref[...]
    h = jnp.maximum(h, 0.0)
    o = jnp.dot(h, w3_ref[...], preferred_element_type=hp) + b3_ref[...]
    o = o + jnp.dot(ex, p_ref[...], preferred_element_type=hp)
    o_ref[...] = o


def _mlp(g, xr, w1m, w1e, b1, w2, b2, w3e, b3e, p):
    return pl.pallas_call(
        _mlp_body,
        grid=(TILES,),
        in_specs=[
            pl.BlockSpec((TILE, NF * DIM), lambda i: (i, 0)),
            pl.BlockSpec((TILE, NF + 3), lambda i: (i, 0)),
            pl.BlockSpec((NF * DIM, 128), lambda i: (0, 0)),
            pl.BlockSpec((NF + 3, 128), lambda i: (0, 0)),
            pl.BlockSpec((1, 128), lambda i: (0, 0)),
            pl.BlockSpec((128, 128), lambda i: (0, 0)),
            pl.BlockSpec((1, 128), lambda i: (0, 0)),
            pl.BlockSpec((128, OUT_DIM), lambda i: (0, 0)),
            pl.BlockSpec((1, OUT_DIM), lambda i: (0, 0)),
            pl.BlockSpec((NF + 3, OUT_DIM), lambda i: (0, 0)),
        ],
        out_specs=pl.BlockSpec((TILE, OUT_DIM), lambda i: (i, 0)),
        out_shape=jax.ShapeDtypeStruct((N, OUT_DIM), jnp.float32),
    )(g, xr, w1m, w1e, b1, w2, b2, w3e, b3e, p)


def kernel(x, table_0, table_1, table_2, table_3, W1, b1, W2, b2, W3, b3):
    xr = x.reshape(N, NF + 3)
    idx = xr[:, :NF].astype(jnp.int32).T.reshape(NF, NROWS, LANES)

    g = _sc_gather(table_0, table_1, table_2, table_3, idx)

    w1m = W1[: NF * DIM]
    w1e = jnp.concatenate(
        [jnp.zeros((NF, 128), jnp.float32), W1[NF * DIM:]], axis=0)
    w3e = jnp.concatenate([W3, jnp.zeros((128, 1), jnp.float32)], axis=1)
    b3e = jnp.concatenate([b3, jnp.zeros((1,), jnp.float32)])[None, :]
    p = jnp.zeros((NF + 3, OUT_DIM), jnp.float32).at[NF + 2, OUT_DIM - 1].set(1.0)

    out = _mlp(g, xr, w1m, w1e, b1[None, :], W2, b2[None, :], w3e, b3e, p)
    return out.reshape(B, L, OUT_DIM)


# speedup vs baseline: 1.5791x; 1.4711x over previous
import functools

import jax
import jax.numpy as jnp
from jax import lax
from jax.experimental import pallas as pl
from jax.experimental.pallas import tpu as pltpu
import jax.experimental.pallas.tpu_sc as plsc

B, L, NF = 4096, 50, 4
VOCAB, DIM = 100000, 32
OUT_DIM = NF * DIM + 3
N = B * L
LANES = 128
NROWS = N // LANES
NW = 32
ROWS_PER_W = NROWS // NW
TOK_PER_W = N // NW
GROUP = 5
CHUNK_TOK = GROUP * LANES
NCHUNK = ROWS_PER_W // GROUP


def _sc_gather_body(t0, t1, t2, t3, idx, g_hbm, idx_v, buf_v, sem):
    tables = (t0, t1, t2, t3)
    wid = lax.axis_index("s") * 2 + lax.axis_index("c")
    base_tok = wid * TOK_PER_W

    for f in range(NF):
        pltpu.sync_copy(idx.at[f, pl.ds(wid * ROWS_PER_W, ROWS_PER_W)],
                        idx_v.at[f])

    def chunk(c, carry):
        copies = []
        for f in range(NF):
            for j in range(GROUP):
                r = c * GROUP + j
                copies.append(
                    pltpu.async_copy(
                        tables[f].at[idx_v.at[f, r]],
                        buf_v.at[f, pl.ds(j * LANES, LANES)],
                        sem,
                    )
                )
        for cp in copies:
            cp.wait()
        for f in range(NF):
            pltpu.sync_copy(
                buf_v.at[f],
                g_hbm.at[pl.ds(base_tok + c * CHUNK_TOK, CHUNK_TOK),
                         pl.ds(f * DIM, DIM)],
            )
        return carry

    lax.fori_loop(0, NCHUNK, chunk, 0)


_sc_gather = functools.partial(
    pl.kernel,
    mesh=plsc.VectorSubcoreMesh(core_axis_name="c", subcore_axis_name="s"),
    compiler_params=pltpu.CompilerParams(use_tc_tiling_on_sc=False),
    out_type=jax.ShapeDtypeStruct((N, NF * DIM), jnp.float32),
    scratch_types=[
        pltpu.VMEM((NF, ROWS_PER_W, LANES), jnp.int32),
        pltpu.VMEM((NF, CHUNK_TOK, DIM), jnp.float32),
        pltpu.SemaphoreType.DMA,
    ],
)(_sc_gather_body)


TB = 64
TILE = TB * L
TILES = B // TB


def _mlp_body(g_ref, xt_ref, w1m_ref, w1f_ref, b1_ref, w2_ref, b2_ref,
              w3_ref, b3_ref, p_ref, o_ref):
    hp = jnp.float32
    g = g_ref[...]
    xt = xt_ref[...]
    dn = (((0,), (0,)), ((), ()))
    h = jnp.dot(g, w1m_ref[...], preferred_element_type=hp)
    h = h + lax.dot_general(xt, w1f_ref[...], dn, preferred_element_type=hp)
    h = jnp.maximum(h + b1_ref[...], 0.0)
    h = jnp.dot(h, w2_ref[...], preferred_element_type=hp) + b2_ref[...]
    h = jnp.maximum(h, 0.0)
    o = jnp.dot(h, w3_ref[...], preferred_element_type=hp) + b3_ref[...]
    o = o + lax.dot_general(xt, p_ref[...], dn, preferred_element_type=hp)
    o_ref[...] = o.reshape(TB, L, OUT_DIM)


def _mlp(g, xt, w1m, w1f, b1, w2, b2, w3e, b3e, p):
    return pl.pallas_call(
        _mlp_body,
        grid=(TILES,),
        in_specs=[
            pl.BlockSpec((TILE, NF * DIM), lambda i: (i, 0)),
            pl.BlockSpec((3, TILE), lambda i: (0, i)),
            pl.BlockSpec((NF * DIM, 128), lambda i: (0, 0)),
            pl.BlockSpec((3, 128), lambda i: (0, 0)),
            pl.BlockSpec((1, 128), lambda i: (0, 0)),
            pl.BlockSpec((128, 128), lambda i: (0, 0)),
            pl.BlockSpec((1, 128), lambda i: (0, 0)),
            pl.BlockSpec((128, OUT_DIM), lambda i: (0, 0)),
            pl.BlockSpec((1, OUT_DIM), lambda i: (0, 0)),
            pl.BlockSpec((3, OUT_DIM), lambda i: (0, 0)),
        ],
        out_specs=pl.BlockSpec((TB, L, OUT_DIM), lambda i: (i, 0, 0)),
        out_shape=jax.ShapeDtypeStruct((B, L, OUT_DIM), jnp.float32),
    )(g, xt, w1m, w1f, b1, w2, b2, w3e, b3e, p)


def kernel(x, table_0, table_1, table_2, table_3, W1, b1, W2, b2, W3, b3):
    xrT = x.reshape(N, NF + 3).T
    idx = xrT[:NF].astype(jnp.int32).reshape(NF, NROWS, LANES)
    xt = xrT[NF:]

    g = _sc_gather(table_0, table_1, table_2, table_3, idx)

    w1m = W1[: NF * DIM]
    w1f = W1[NF * DIM:]
    w3e = jnp.concatenate([W3, jnp.zeros((128, 1), jnp.float32)], axis=1)
    b3e = jnp.concatenate([b3, jnp.zeros((1,), jnp.float32)])[None, :]
    p = jnp.zeros((3, OUT_DIM), jnp.float32).at[2, OUT_DIM - 1].set(1.0)

    return _mlp(g, xt, w1m, w1f, b1[None, :], W2, b2[None, :], w3e, b3e, p)


# speedup vs baseline: 1.6204x; 1.0261x over previous
import functools

import jax
import jax.numpy as jnp
from jax import lax
from jax.experimental import pallas as pl
from jax.experimental.pallas import tpu as pltpu
import jax.experimental.pallas.tpu_sc as plsc

B, L, NF = 4096, 50, 4
VOCAB, DIM = 100000, 32
OUT_DIM = NF * DIM + 3
N = B * L
LANES = 128
NROWS = N // LANES
NW = 32
ROWS_PER_W = NROWS // NW
TOK_PER_W = N // NW
GROUP = 5
CHUNK_TOK = GROUP * LANES
NCHUNK = ROWS_PER_W // GROUP


def _sc_gather_body(t0, t1, t2, t3, idx, g_hbm, idx_v, buf_v, sem):
    tables = (t0, t1, t2, t3)
    wid = lax.axis_index("s") * 2 + lax.axis_index("c")
    base_tok = wid * TOK_PER_W

    for f in range(NF):
        pltpu.sync_copy(idx.at[f, pl.ds(wid * ROWS_PER_W, ROWS_PER_W)],
                        idx_v.at[f])

    def chunk(c, carry):
        copies = []
        for f in range(NF):
            for j in range(GROUP):
                r = c * GROUP + j
                copies.append(
                    pltpu.async_copy(
                        tables[f].at[idx_v.at[f, r]],
                        buf_v.at[f, pl.ds(j * LANES, LANES)],
                        sem,
                    )
                )
        for cp in copies:
            cp.wait()
        for f in range(NF):
            pltpu.sync_copy(
                buf_v.at[f],
                g_hbm.at[pl.ds(base_tok + c * CHUNK_TOK, CHUNK_TOK),
                         pl.ds(f * DIM, DIM)],
            )
        return carry

    lax.fori_loop(0, NCHUNK, chunk, 0)


_sc_gather = functools.partial(
    pl.kernel,
    mesh=plsc.VectorSubcoreMesh(core_axis_name="c", subcore_axis_name="s"),
    compiler_params=pltpu.CompilerParams(use_tc_tiling_on_sc=False),
    out_type=jax.ShapeDtypeStruct((N, NF * DIM), jnp.float32),
    scratch_types=[
        pltpu.VMEM((NF, ROWS_PER_W, LANES), jnp.int32),
        pltpu.VMEM((NF, CHUNK_TOK, DIM), jnp.float32),
        pltpu.SemaphoreType.DMA,
    ],
)(_sc_gather_body)


TB = 128
TILE = TB * L
TILES = B // TB


def _mlp_body(g_ref, xt_ref, w1m_ref, w1f_ref, b1_ref, w2_ref, b2_ref,
              w3_ref, b3_ref, p_ref, o_ref):
    hp = jnp.float32
    g = g_ref[...]
    xt = xt_ref[...]
    dn = (((0,), (0,)), ((), ()))
    h = jnp.dot(g, w1m_ref[...], preferred_element_type=hp)
    h = h + lax.dot_general(xt, w1f_ref[...], dn, preferred_element_type=hp)
    h = jnp.maximum(h + b1_ref[...], 0.0)
    h = jnp.dot(h, w2_ref[...], preferred_element_type=hp) + b2_ref[...]
    h = jnp.maximum(h, 0.0)
    o = jnp.dot(h, w3_ref[...], preferred_element_type=hp) + b3_ref[...]
    o = o + lax.dot_general(xt, p_ref[...], dn, preferred_element_type=hp)
    o_ref[...] = o.reshape(TB, L, OUT_DIM)


def _mlp(g, xt, w1m, w1f, b1, w2, b2, w3e, b3e, p):
    return pl.pallas_call(
        _mlp_body,
        grid=(TILES,),
        in_specs=[
            pl.BlockSpec((TILE, NF * DIM), lambda i: (i, 0)),
            pl.BlockSpec((3, TILE), lambda i: (0, i)),
            pl.BlockSpec((NF * DIM, 128), lambda i: (0, 0)),
            pl.BlockSpec((3, 128), lambda i: (0, 0)),
            pl.BlockSpec((1, 128), lambda i: (0, 0)),
            pl.BlockSpec((128, 128), lambda i: (0, 0)),
            pl.BlockSpec((1, 128), lambda i: (0, 0)),
            pl.BlockSpec((128, OUT_DIM), lambda i: (0, 0)),
            pl.BlockSpec((1, OUT_DIM), lambda i: (0, 0)),
            pl.BlockSpec((3, OUT_DIM), lambda i: (0, 0)),
        ],
        out_specs=pl.BlockSpec((TB, L, OUT_DIM), lambda i: (i, 0, 0)),
        out_shape=jax.ShapeDtypeStruct((B, L, OUT_DIM), jnp.float32),
    )(g, xt, w1m, w1f, b1, w2, b2, w3e, b3e, p)


def kernel(x, table_0, table_1, table_2, table_3, W1, b1, W2, b2, W3, b3):
    xrT = x.reshape(N, NF + 3).T
    idx = xrT[:NF].astype(jnp.int32).reshape(NF, NROWS, LANES)
    xt = xrT[NF:]

    g = _sc_gather(table_0, table_1, table_2, table_3, idx)

    w1m = W1[: NF * DIM]
    w1f = W1[NF * DIM:]
    w3e = jnp.concatenate([W3, jnp.zeros((128, 1), jnp.float32)], axis=1)
    b3e = jnp.concatenate([b3, jnp.zeros((1,), jnp.float32)])[None, :]
    p = jnp.zeros((3, OUT_DIM), jnp.float32).at[2, OUT_DIM - 1].set(1.0)

    return _mlp(g, xt, w1m, w1f, b1[None, :], W2, b2[None, :], w3e, b3e, p)


# speedup vs baseline: 1.6248x; 1.0028x over previous
import functools

import jax
import jax.numpy as jnp
from jax import lax
from jax.experimental import pallas as pl
from jax.experimental.pallas import tpu as pltpu
import jax.experimental.pallas.tpu_sc as plsc

B, L, NF = 4096, 50, 4
VOCAB, DIM = 100000, 32
OUT_DIM = NF * DIM + 3
N = B * L
LANES = 128
NROWS = N // LANES
NW = 32
ROWS_PER_W = NROWS // NW
TOK_PER_W = N // NW
GROUP = 5
CHUNK_TOK = GROUP * LANES
NCHUNK = ROWS_PER_W // GROUP


def _sc_gather_body(t0, t1, t2, t3, idx, g_hbm, idx_v, buf_v, sem):
    tables = (t0, t1, t2, t3)
    wid = lax.axis_index("s") * 2 + lax.axis_index("c")
    base_tok = wid * TOK_PER_W

    for f in range(NF):
        pltpu.sync_copy(idx.at[f, pl.ds(wid * ROWS_PER_W, ROWS_PER_W)],
                        idx_v.at[f])

    def chunk(c, carry):
        copies = []
        for f in range(NF):
            for j in range(GROUP):
                r = c * GROUP + j
                copies.append(
                    pltpu.async_copy(
                        tables[f].at[idx_v.at[f, r]],
                        buf_v.at[f, pl.ds(j * LANES, LANES)],
                        sem,
                    )
                )
        for cp in copies:
            cp.wait()
        for f in range(NF):
            pltpu.sync_copy(
                buf_v.at[f],
                g_hbm.at[pl.ds(base_tok + c * CHUNK_TOK, CHUNK_TOK),
                         pl.ds(f * DIM, DIM)],
            )
        return carry

    lax.fori_loop(0, NCHUNK, chunk, 0)


_sc_gather = functools.partial(
    pl.kernel,
    mesh=plsc.VectorSubcoreMesh(core_axis_name="c", subcore_axis_name="s"),
    compiler_params=pltpu.CompilerParams(use_tc_tiling_on_sc=False),
    out_type=jax.ShapeDtypeStruct((N, NF * DIM), jnp.float32),
    scratch_types=[
        pltpu.VMEM((NF, ROWS_PER_W, LANES), jnp.int32),
        pltpu.VMEM((NF, CHUNK_TOK, DIM), jnp.float32),
        pltpu.SemaphoreType.DMA,
    ],
)(_sc_gather_body)


TB = 256
TILE = TB * L
TILES = B // TB


def _mlp_body(g_ref, xt_ref, w1m_ref, w1f_ref, b1_ref, w2_ref, b2_ref,
              w3_ref, b3_ref, p_ref, o_ref):
    hp = jnp.float32
    g = g_ref[...]
    xt = xt_ref[...]
    dn = (((0,), (0,)), ((), ()))
    h = jnp.dot(g, w1m_ref[...], preferred_element_type=hp)
    h = h + lax.dot_general(xt, w1f_ref[...], dn, preferred_element_type=hp)
    h = jnp.maximum(h + b1_ref[...], 0.0)
    h = jnp.dot(h, w2_ref[...], preferred_element_type=hp) + b2_ref[...]
    h = jnp.maximum(h, 0.0)
    o = jnp.dot(h, w3_ref[...], preferred_element_type=hp) + b3_ref[...]
    o = o + lax.dot_general(xt, p_ref[...], dn, preferred_element_type=hp)
    o_ref[...] = o.reshape(TB, L, OUT_DIM)


def _mlp(g, xt, w1m, w1f, b1, w2, b2, w3e, b3e, p):
    return pl.pallas_call(
        _mlp_body,
        grid=(TILES,),
        in_specs=[
            pl.BlockSpec((TILE, NF * DIM), lambda i: (i, 0)),
            pl.BlockSpec((3, TILE), lambda i: (0, i)),
            pl.BlockSpec((NF * DIM, 128), lambda i: (0, 0)),
            pl.BlockSpec((3, 128), lambda i: (0, 0)),
            pl.BlockSpec((1, 128), lambda i: (0, 0)),
            pl.BlockSpec((128, 128), lambda i: (0, 0)),
            pl.BlockSpec((1, 128), lambda i: (0, 0)),
            pl.BlockSpec((128, OUT_DIM), lambda i: (0, 0)),
            pl.BlockSpec((1, OUT_DIM), lambda i: (0, 0)),
            pl.BlockSpec((3, OUT_DIM), lambda i: (0, 0)),
        ],
        out_specs=pl.BlockSpec((TB, L, OUT_DIM), lambda i: (i, 0, 0)),
        out_shape=jax.ShapeDtypeStruct((B, L, OUT_DIM), jnp.float32),
    )(g, xt, w1m, w1f, b1, w2, b2, w3e, b3e, p)


def kernel(x, table_0, table_1, table_2, table_3, W1, b1, W2, b2, W3, b3):
    xrT = x.reshape(N, NF + 3).T
    idx = xrT[:NF].astype(jnp.int32).reshape(NF, NROWS, LANES)
    xt = xrT[NF:]

    g = _sc_gather(table_0, table_1, table_2, table_3, idx)

    w1m = W1[: NF * DIM]
    w1f = W1[NF * DIM:]
    w3e = jnp.concatenate([W3, jnp.zeros((128, 1), jnp.float32)], axis=1)
    b3e = jnp.concatenate([b3, jnp.zeros((1,), jnp.float32)])[None, :]
    p = jnp.zeros((3, OUT_DIM), jnp.float32).at[2, OUT_DIM - 1].set(1.0)

    return _mlp(g, xt, w1m, w1f, b1[None, :], W2, b2[None, :], w3e, b3e, p)
